# baseline (device time: 313056 ns/iter reference)
import functools

import jax
import jax.numpy as jnp
from jax import lax
from jax.experimental import pallas as pl
from jax.experimental.pallas import tpu as pltpu

N_DEV = 4
AXIS = "i"


def _ag_body(x_ref, o_ref, send_r, recv_r, send_l, recv_l):
    my_id = lax.axis_index(AXIS)
    right = lax.rem(my_id + 1, N_DEV)
    left = lax.rem(my_id + N_DEV - 1, N_DEV)
    m_half = x_ref.shape[0] // 2

    o_ref[0] = x_ref[:m_half].astype(jnp.bfloat16)
    o_ref[1] = x_ref[m_half:].astype(jnp.bfloat16)

    barrier = pltpu.get_barrier_semaphore()
    pl.semaphore_signal(barrier, 1, device_id=(left,))
    pl.semaphore_signal(barrier, 1, device_id=(right,))
    pl.semaphore_wait(barrier, 2)

    dma_r = dma_l = None
    for h in range(N_DEV - 1):
        if h:
            dma_r.wait()
            dma_l.wait()
        dma_r = pltpu.make_async_remote_copy(
            src_ref=o_ref.at[1 if h == 0 else 1 + h],
            dst_ref=o_ref.at[2 + h],
            send_sem=send_r.at[h],
            recv_sem=recv_r.at[h],
            device_id=(right,),
        )
        dma_r.start()
        dma_l = pltpu.make_async_remote_copy(
            src_ref=o_ref.at[0 if h == 0 else 4 + h],
            dst_ref=o_ref.at[5 + h],
            send_sem=send_l.at[h],
            recv_sem=recv_l.at[h],
            device_id=(left,),
        )
        dma_l.start()
    dma_r.wait()
    dma_l.wait()


def _all_gather(x_f32):
    m_per, k = x_f32.shape
    return pl.pallas_call(
        _ag_body,
        out_shape=jax.ShapeDtypeStruct(
            (2 * N_DEV, m_per // 2, k), jnp.bfloat16
        ),
        in_specs=[pl.BlockSpec(memory_space=pltpu.MemorySpace.VMEM)],
        out_specs=pl.BlockSpec(memory_space=pltpu.MemorySpace.VMEM),
        scratch_shapes=[
            pltpu.SemaphoreType.DMA((N_DEV - 1,)),
            pltpu.SemaphoreType.DMA((N_DEV - 1,)),
            pltpu.SemaphoreType.DMA((N_DEV - 1,)),
            pltpu.SemaphoreType.DMA((N_DEV - 1,)),
        ],
        compiler_params=pltpu.CompilerParams(collective_id=0),
    )(x_f32)


def _gemm_body(perm_ref, x_ref, w_ref, o_ref, amax_ref):
    m = pl.program_id(0)

    o_ref[...] = jnp.dot(
        x_ref[0], w_ref[...], preferred_element_type=jnp.float32
    )

    @pl.when(m == 0)
    def _():
        amax_ref[0, 0] = 0.0

    amax_ref[0, 0] = jnp.maximum(amax_ref[0, 0], jnp.max(jnp.abs(o_ref[...])))


def _gemm(x_rot, w_bf16, perm):
    n_slots, bm, k = x_rot.shape
    _, n = w_bf16.shape
    m = n_slots * bm
    y, amax = pl.pallas_call(
        _gemm_body,
        grid_spec=pltpu.PrefetchScalarGridSpec(
            num_scalar_prefetch=1,
            grid=(n_slots,),
            in_specs=[
                pl.BlockSpec((1, bm, k), lambda i, perm: (perm[i], 0, 0)),
                pl.BlockSpec(memory_space=pltpu.MemorySpace.VMEM),
            ],
            out_specs=[
                pl.BlockSpec((bm, n), lambda i, perm: (i, 0)),
                pl.BlockSpec(memory_space=pltpu.MemorySpace.SMEM),
            ],
        ),
        out_shape=[
            jax.ShapeDtypeStruct((m, n), jnp.float32),
            jax.ShapeDtypeStruct((1, 1), jnp.float32),
        ],
        compiler_params=pltpu.CompilerParams(
            dimension_semantics=("arbitrary",),
            vmem_limit_bytes=60 * 1024 * 1024,
        ),
    )(perm, x_rot, w_bf16)
    return y, amax


def _epilogue_body(
    y_ref, a_ref, o_ref, scale_ref, comm_ref, send_sems, recv_sems
):
    i = pl.program_id(0)

    @pl.when(i == 0)
    def _():
        my_id = lax.axis_index(AXIS)
        right = lax.rem(my_id + 1, N_DEV)
        left = lax.rem(my_id + N_DEV - 1, N_DEV)

        comm_ref[0] = jnp.full((8, 128), a_ref[0, 0], dtype=jnp.float32)

        barrier = pltpu.get_barrier_semaphore()
        pl.semaphore_signal(barrier, 1, device_id=(left,))
        pl.semaphore_signal(barrier, 1, device_id=(right,))
        pl.semaphore_wait(barrier, 2)

        for h in range(N_DEV - 1):
            dma = pltpu.make_async_remote_copy(
                src_ref=comm_ref.at[h],
                dst_ref=comm_ref.at[h + 1],
                send_sem=send_sems.at[h],
                recv_sem=recv_sems.at[h],
                device_id=(right,),
            )
            dma.start()
            dma.wait()

        scale_ref[0, 0] = jnp.max(comm_ref[...]) / 127.0

    scale = scale_ref[0, 0]
    q = jnp.clip(jnp.round(y_ref[...] / scale), -127.0, 127.0)
    o_ref[...] = q * scale


def _epilogue(y, amax_local, bm=512):
    m, n = y.shape
    return pl.pallas_call(
        _epilogue_body,
        grid=(m // bm,),
        in_specs=[
            pl.BlockSpec((bm, n), lambda i: (i, 0)),
            pl.BlockSpec(memory_space=pltpu.MemorySpace.SMEM),
        ],
        out_specs=pl.BlockSpec((bm, n), lambda i: (i, 0)),
        out_shape=jax.ShapeDtypeStruct((m, n), jnp.float32),
        scratch_shapes=[
            pltpu.SMEM((1, 1), jnp.float32),
            pltpu.VMEM((N_DEV, 8, 128), jnp.float32),
            pltpu.SemaphoreType.DMA((N_DEV - 1,)),
            pltpu.SemaphoreType.DMA((N_DEV - 1,)),
        ],
        compiler_params=pltpu.CompilerParams(
            collective_id=1, dimension_semantics=("arbitrary",)
        ),
    )(y, amax_local)


def kernel(x, w_mat):
    x_rot = _all_gather(x)
    my_id = lax.axis_index(AXIS)
    g = jnp.arange(2 * N_DEV, dtype=jnp.int32)
    c = g // 2
    half = g % 2
    h_hi = jnp.mod(my_id - c - 1, N_DEV)
    h_lo = jnp.mod(c - my_id - 1, N_DEV)
    perm = jnp.where(
        c == my_id, half, jnp.where(half == 1, 2 + h_hi, 5 + h_lo)
    ).astype(jnp.int32)
    y, amax_local = _gemm(x_rot, w_mat.astype(jnp.bfloat16), perm)
    return _epilogue(y, amax_local)


# device time: 294087 ns/iter; 1.0645x vs baseline; 1.0645x over previous
import functools

import jax
import jax.numpy as jnp
from jax import lax
from jax.experimental import pallas as pl
from jax.experimental.pallas import tpu as pltpu

N_DEV = 4
AXIS = "i"


def _ag_body(x_ref, o_ref, send_r, recv_r, send_l, recv_l):
    my_id = lax.axis_index(AXIS)
    right = lax.rem(my_id + 1, N_DEV)
    left = lax.rem(my_id + N_DEV - 1, N_DEV)
    m_half = x_ref.shape[0] // 2

    o_ref[0] = x_ref[:m_half]
    o_ref[1] = x_ref[m_half:]

    barrier = pltpu.get_barrier_semaphore()
    pl.semaphore_signal(barrier, 1, device_id=(left,))
    pl.semaphore_signal(barrier, 1, device_id=(right,))
    pl.semaphore_wait(barrier, 2)

    dma_r = dma_l = None
    for h in range(N_DEV - 1):
        if h:
            dma_r.wait()
            dma_l.wait()
        dma_r = pltpu.make_async_remote_copy(
            src_ref=o_ref.at[1 if h == 0 else 1 + h],
            dst_ref=o_ref.at[2 + h],
            send_sem=send_r.at[h],
            recv_sem=recv_r.at[h],
            device_id=(right,),
        )
        dma_r.start()
        dma_l = pltpu.make_async_remote_copy(
            src_ref=o_ref.at[0 if h == 0 else 4 + h],
            dst_ref=o_ref.at[5 + h],
            send_sem=send_l.at[h],
            recv_sem=recv_l.at[h],
            device_id=(left,),
        )
        dma_l.start()
    dma_r.wait()
    dma_l.wait()


def _all_gather(x_bf16):
    m_per, k = x_bf16.shape
    return pl.pallas_call(
        _ag_body,
        out_shape=jax.ShapeDtypeStruct(
            (2 * N_DEV, m_per // 2, k), x_bf16.dtype
        ),
        in_specs=[pl.BlockSpec(memory_space=pltpu.MemorySpace.VMEM)],
        out_specs=pl.BlockSpec(memory_space=pltpu.MemorySpace.VMEM),
        scratch_shapes=[
            pltpu.SemaphoreType.DMA((N_DEV - 1,)),
            pltpu.SemaphoreType.DMA((N_DEV - 1,)),
            pltpu.SemaphoreType.DMA((N_DEV - 1,)),
            pltpu.SemaphoreType.DMA((N_DEV - 1,)),
        ],
        compiler_params=pltpu.CompilerParams(collective_id=0),
    )(x_bf16)


def _gemm_body(perm_ref, x_ref, w_ref, o_ref, amax_ref):
    m = pl.program_id(0)

    o_ref[...] = jnp.dot(
        x_ref[0], w_ref[...], preferred_element_type=jnp.float32
    )

    @pl.when(m == 0)
    def _():
        amax_ref[0, 0] = 0.0

    amax_ref[0, 0] = jnp.maximum(amax_ref[0, 0], jnp.max(jnp.abs(o_ref[...])))


def _gemm(x_rot, w_bf16, perm):
    n_slots, bm, k = x_rot.shape
    _, n = w_bf16.shape
    m = n_slots * bm
    y, amax = pl.pallas_call(
        _gemm_body,
        grid_spec=pltpu.PrefetchScalarGridSpec(
            num_scalar_prefetch=1,
            grid=(n_slots,),
            in_specs=[
                pl.BlockSpec((1, bm, k), lambda i, perm: (perm[i], 0, 0)),
                pl.BlockSpec(memory_space=pltpu.MemorySpace.VMEM),
            ],
            out_specs=[
                pl.BlockSpec((bm, n), lambda i, perm: (i, 0)),
                pl.BlockSpec(memory_space=pltpu.MemorySpace.SMEM),
            ],
        ),
        out_shape=[
            jax.ShapeDtypeStruct((m, n), jnp.float32),
            jax.ShapeDtypeStruct((1, 1), jnp.float32),
        ],
        compiler_params=pltpu.CompilerParams(
            dimension_semantics=("arbitrary",),
            vmem_limit_bytes=60 * 1024 * 1024,
        ),
    )(perm, x_rot, w_bf16)
    return y, amax


def _armax_body(a_ref, o_ref, comm_ref, local_sem, send_sems, recv_sems):
    my_id = lax.axis_index(AXIS)
    right = lax.rem(my_id + 1, N_DEV)
    left = lax.rem(my_id + N_DEV - 1, N_DEV)

    comm_ref[0] = jnp.full((8, 128), a_ref[0, 0], dtype=jnp.float32)

    barrier = pltpu.get_barrier_semaphore()
    pl.semaphore_signal(barrier, 1, device_id=(left,))
    pl.semaphore_signal(barrier, 1, device_id=(right,))
    pl.semaphore_wait(barrier, 2)

    for h in range(N_DEV - 1):
        dma = pltpu.make_async_remote_copy(
            src_ref=comm_ref.at[h],
            dst_ref=comm_ref.at[h + 1],
            send_sem=send_sems.at[h],
            recv_sem=recv_sems.at[h],
            device_id=(right,),
        )
        dma.start()
        dma.wait()

    o_ref[0, 0] = jnp.max(comm_ref[...])


def _armax(amax_local):
    return pl.pallas_call(
        _armax_body,
        out_shape=jax.ShapeDtypeStruct((1, 1), jnp.float32),
        in_specs=[pl.BlockSpec(memory_space=pltpu.MemorySpace.SMEM)],
        out_specs=pl.BlockSpec(memory_space=pltpu.MemorySpace.SMEM),
        scratch_shapes=[
            pltpu.VMEM((N_DEV, 8, 128), jnp.float32),
            pltpu.SemaphoreType.DMA,
            pltpu.SemaphoreType.DMA((N_DEV - 1,)),
            pltpu.SemaphoreType.DMA((N_DEV - 1,)),
        ],
        compiler_params=pltpu.CompilerParams(collective_id=1),
    )(amax_local)


def _epilogue_body(y_ref, amax_ref, o_ref):
    scale = amax_ref[0, 0] / 127.0
    q = jnp.clip(jnp.round(y_ref[...] / scale), -127.0, 127.0)
    o_ref[...] = q * scale


def _epilogue(y, amax, bm=512):
    m, n = y.shape
    return pl.pallas_call(
        _epilogue_body,
        grid=(m // bm,),
        in_specs=[
            pl.BlockSpec((bm, n), lambda i: (i, 0)),
            pl.BlockSpec(memory_space=pltpu.MemorySpace.SMEM),
        ],
        out_specs=pl.BlockSpec((bm, n), lambda i: (i, 0)),
        out_shape=jax.ShapeDtypeStruct((m, n), jnp.float32),
    )(y, amax)


def kernel(x, w_mat):
    x_rot = _all_gather(x.astype(jnp.bfloat16))
    my_id = lax.axis_index(AXIS)
    g = jnp.arange(2 * N_DEV, dtype=jnp.int32)
    c = g // 2
    half = g % 2
    h_hi = jnp.mod(my_id - c - 1, N_DEV)
    h_lo = jnp.mod(c - my_id - 1, N_DEV)
    perm = jnp.where(
        c == my_id, half, jnp.where(half == 1, 2 + h_hi, 5 + h_lo)
    ).astype(jnp.int32)
    y, amax_local = _gemm(x_rot, w_mat.astype(jnp.bfloat16), perm)
    amax = _armax(amax_local)
    return _epilogue(y, amax)
